# baseline (speedup 1.0000x reference)
import functools
import jax
import jax.numpy as jnp
from jax import lax
from jax.experimental import pallas as pl
from jax.experimental.pallas import tpu as pltpu
from jax.experimental.pallas import tpu_sc as plsc

_S, _B, _D = 2048, 4, 1024
_NC, _NS = 2, 16
_NW = _NC * _NS
_ROWS_PER_W = _S // _NW
_C = 8
_NT = _ROWS_PER_W // _C
_NBUF = 3
_L = 16


def _sc_body(x_hbm, emb_hbm, out_hbm, x_v, e_v, in_sems, out_sems):
    wid = lax.axis_index("s") * _NC + lax.axis_index("c")
    base = wid * _ROWS_PER_W

    def start_in(k):
        row0 = base + k * _C
        buf = k % _NBUF
        sem = in_sems.at[buf]
        return (
            pltpu.async_copy(x_hbm.at[pl.ds(row0, _C)], x_v.at[buf], sem),
            pltpu.async_copy(emb_hbm.at[pl.ds(row0, _C)], e_v.at[buf], sem),
        )

    def start_out(k):
        row0 = base + k * _C
        buf = k % _NBUF
        return pltpu.async_copy(
            x_v.at[buf], out_hbm.at[pl.ds(row0, _C)], out_sems.at[buf]
        )

    in_d = {}
    out_d = {}
    in_d[0] = start_in(0)
    in_d[1] = start_in(1)
    for k in range(_NT):
        buf = k % _NBUF
        for d in in_d.pop(k):
            d.wait()

        @plsc.parallel_loop(0, _D, step=_L, unroll=4)
        def _(j):
            dcol = pl.ds(j, _L)
            for si in range(_C):
                e = e_v[buf, si, dcol]
                for b in range(_B):
                    x_v[buf, si, b, dcol] = x_v[buf, si, b, dcol] + e

        out_d[k] = start_out(k)
        if k + 2 < _NT:
            if k - 1 >= 0:
                out_d.pop(k - 1).wait()
            in_d[k + 2] = start_in(k + 2)
    for k in sorted(out_d):
        out_d.pop(k).wait()


def kernel(x, emb):
    mesh = plsc.VectorSubcoreMesh(core_axis_name="c", subcore_axis_name="s")
    run = functools.partial(
        pl.kernel,
        out_type=jax.ShapeDtypeStruct((_S, _B, _D), jnp.float32),
        mesh=mesh,
        scratch_types=[
            pltpu.VMEM((_NBUF, _C, _B, _D), jnp.float32),
            pltpu.VMEM((_NBUF, _C, _D), jnp.float32),
            pltpu.SemaphoreType.DMA((_NBUF,)),
            pltpu.SemaphoreType.DMA((_NBUF,)),
        ],
    )(_sc_body)
    return run(x, emb)


# speedup vs baseline: 2.2855x; 2.2855x over previous
import jax
import jax.numpy as jnp
from jax.experimental import pallas as pl

_BLK_S = 512


def _pe_add_kernel(x_ref, emb_ref, o_ref):
    o_ref[...] = x_ref[...] + emb_ref[...][:, None, :]


def kernel(x, emb):
    S, B, D = x.shape
    grid = (S // _BLK_S,)
    return pl.pallas_call(
        _pe_add_kernel,
        grid=grid,
        in_specs=[
            pl.BlockSpec((_BLK_S, B, D), lambda i: (i, 0, 0)),
            pl.BlockSpec((_BLK_S, D), lambda i: (i, 0)),
        ],
        out_specs=pl.BlockSpec((_BLK_S, B, D), lambda i: (i, 0, 0)),
        out_shape=jax.ShapeDtypeStruct((S, B, D), x.dtype),
    )(x, emb)
